# baseline (speedup 1.0000x reference)
import functools

import jax
import jax.numpy as jnp
from jax import lax
from jax.experimental import pallas as pl
from jax.experimental.pallas import tpu as pltpu, tpu_sc as plsc

B = 16384
F = 26
D = 32
NC = 2
NS = 16
NW = NC * NS
L = 16

TABLE_ROWS = 100000 * F
ROWS = B * F
RPW = ROWS // NW
IRPW = RPW // 128
PATR = 13
C = 128
NCH = RPW // C
OC = C // 4


def _gbody(idx_hbm, pat_hbm, table_hbm, out_hbm,
           idx_v, pat_v, buf0, buf1, obuf, gsem0, gsem1):
    wid = lax.axis_index("s") * NC + lax.axis_index("c")
    ibase = pl.multiple_of(wid * IRPW, 8)
    obase = pl.multiple_of(wid * (RPW // 4), 8)

    pltpu.sync_copy(idx_hbm.at[pl.ds(ibase, IRPW)], idx_v)
    pltpu.sync_copy(pat_hbm, pat_v)

    def add_rows(g2, carry):
        for jj in range(PATR):
            i = g2 * PATR + jj
            for j in range(8):
                sl = pl.ds(j * L, L)
                idx_v[i, sl] = idx_v[i, sl] + pat_v[jj, sl]
        return carry

    lax.fori_loop(0, IRPW // PATR, add_rows, 0)

    def start(k, buf, sem):
        return pltpu.async_copy(table_hbm.at[idx_v.at[k]], buf, sem)

    def wait(buf, sem):
        pltpu.make_async_copy(table_hbm.at[pl.ds(0, C)], buf, sem).wait()

    def compact_and_store(k, buf):
        for e in range(C):
            orow = e >> 2
            ocol = (e & 3) * D
            for j in range(2):
                obuf[orow, pl.ds(ocol + j * L, L)] = buf[e, pl.ds(j * L, L)]
        pltpu.sync_copy(obuf, out_hbm.at[pl.ds(obase + k * OC, OC)])

    start(0, buf0, gsem0)

    def loop2(jj, carry):
        k0 = 2 * jj
        k1 = k0 + 1

        @pl.when(k1 < NCH)
        def _():
            start(k1, buf1, gsem1)

        wait(buf0, gsem0)
        compact_and_store(k0, buf0)

        @pl.when(k0 + 2 < NCH)
        def _():
            start(k0 + 2, buf0, gsem0)

        @pl.when(k1 < NCH)
        def _():
            wait(buf1, gsem1)
            compact_and_store(k1, buf1)

        return carry

    lax.fori_loop(0, (NCH + 1) // 2, loop2, 0)


@jax.jit
def _run(idx2d, pat2d, tpad):
    mesh = plsc.VectorSubcoreMesh(core_axis_name="c", subcore_axis_name="s")
    return pl.kernel(
        _gbody,
        out_type=jax.ShapeDtypeStruct((ROWS * D // 128, 128), jnp.float32),
        mesh=mesh,
        scratch_types=[
            pltpu.VMEM((IRPW, 128), jnp.int32),
            pltpu.VMEM((PATR, 128), jnp.int32),
            pltpu.VMEM((C, 128), jnp.float32),
            pltpu.VMEM((C, 128), jnp.float32),
            pltpu.VMEM((OC, 128), jnp.float32),
            pltpu.SemaphoreType.DMA,
            pltpu.SemaphoreType.DMA,
        ],
        compiler_params=pltpu.CompilerParams(
            needs_layout_passes=False, use_tc_tiling_on_sc=True),
    )(idx2d, pat2d, tpad)


def kernel(sparse_indices, offsets, embed_table):
    idx2d = sparse_indices.reshape(ROWS // 128, 128)
    pat2d = jnp.tile(offsets.reshape(F), PATR * 128 // F).reshape(PATR, 128)
    tpad = jnp.pad(embed_table, ((0, 0), (0, 128 - D)))
    out = _run(idx2d, pat2d, tpad)
    return out.reshape(B, F, D)
